# Initial kernel scaffold; baseline (speedup 1.0000x reference)
import math

import jax
import jax.numpy as jnp
from jax import lax
from jax.experimental import pallas as pl
from jax.experimental.pallas import tpu as pltpu
from jax.experimental.pallas import tpu_sc as plsc

NUM_BINS = 64
KERNEL_SIGMA = 1.0
KR = math.ceil(2 * KERNEL_SIGMA)
NBP = NUM_BINS + 2 * KR
EPS = 1e-08

NC = 2
NS = 16
L = 16
NW = NC * NS


def _bsp(d):
    ad = jnp.abs(d)
    ad2 = ad * ad
    ad3 = ad2 * ad
    inner = (3.0 * ad3 - 6.0 * ad2 + 4.0) * (1.0 / 6.0)
    e = 2.0 - ad
    outer = e * e * e * (1.0 / 6.0)
    return jnp.where(ad < 1.0, inner, jnp.where(ad < 2.0, outer, 0.0))


def _minmax_tc(s, t):
    B = s.shape[0]

    def body(s_ref, t_ref, o_ref):
        sv = s_ref[...]
        tv = t_ref[...]
        stats = jnp.concatenate(
            [
                jnp.min(sv, axis=1),
                jnp.max(sv, axis=1),
                jnp.min(tv, axis=1),
                jnp.max(tv, axis=1),
            ]
        )
        o_ref[...] = jnp.broadcast_to(stats[:, None], (4 * B, 128))

    return pl.pallas_call(
        body,
        out_shape=jax.ShapeDtypeStruct((4 * B, 128), jnp.float32),
    )(s, t)


def _sc_hist(s, t, stats):
    B, N = s.shape
    CHUNK = N // NW
    VPB = CHUNK // L

    mesh = plsc.VectorSubcoreMesh(core_axis_name="c", subcore_axis_name="s")

    def body(s_hbm, t_hbm, stats_hbm, jp_hbm, hp_hbm,
             s_v, t_v, stats_v, jp_v, hp_v):
        cid = lax.axis_index("c")
        sid = lax.axis_index("s")
        wid = sid * NC + cid

        pltpu.sync_copy(stats_hbm, stats_v)

        zero = jnp.zeros((L,), jnp.float32)

        def zbody(i, c):
            jp_v[pl.ds(i * L, L)] = zero
            return c

        lax.fori_loop(0, (B * NBP * NBP) // L, zbody, 0)

        def zbody2(i, c):
            hp_v[pl.ds(i * L, L)] = zero
            return c

        lax.fori_loop(0, (2 * B * NBP) // L, zbody2, 0)

        for b in range(B):
            base = wid * CHUNK
            pltpu.sync_copy(s_hbm.at[b, pl.ds(base, CHUNK)], s_v)
            pltpu.sync_copy(t_hbm.at[b, pl.ds(base, CHUNK)], t_v)

            smin = stats_v[0 * B + b, pl.ds(0, L)]
            smax = stats_v[1 * B + b, pl.ds(0, L)]
            tmin = stats_v[2 * B + b, pl.ds(0, L)]
            tmax = stats_v[3 * B + b, pl.ds(0, L)]
            s_bw = (smax - smin) / NUM_BINS
            s_pmin = smin - s_bw * KR
            t_bw = (tmax - tmin) / NUM_BINS
            t_pmin = tmin - t_bw * KR

            jrow0 = b * NBP * NBP
            hrow_s = b * NBP
            hrow_t = (B + b) * NBP

            def vbody(i, c):
                xs = s_v[pl.ds(i * L, L)]
                xt = t_v[pl.ds(i * L, L)]
                bps = (xs - s_pmin) / s_bw
                bpt = (xt - t_pmin) / t_bw
                bis = jnp.clip(bps.astype(jnp.int32), KR, KR + NUM_BINS - 1)
                bit = jnp.clip(bpt.astype(jnp.int32), KR, KR + NUM_BINS - 1)
                bases = bis - (KR - 1)
                baset = bit - (KR - 1)
                fbs = bases.astype(jnp.float32)
                fbt = baset.astype(jnp.float32)
                ws = [_bsp(bps - (fbs + r)) for r in range(2 * KR)]
                wt = [_bsp(bpt - (fbt + r)) for r in range(2 * KR)]
                for r in range(2 * KR):
                    plsc.addupdate_scatter(hp_v, [bases + (hrow_s + r)], ws[r])
                    plsc.addupdate_scatter(hp_v, [baset + (hrow_t + r)], wt[r])
                rowbase = jrow0 + bases * NBP + baset
                for r in range(2 * KR):
                    rb = rowbase + r * NBP
                    for cc in range(2 * KR):
                        plsc.addupdate_scatter(jp_v, [rb + cc], ws[r] * wt[cc])
                return c

            lax.fori_loop(0, VPB, vbody, 0)

        pltpu.sync_copy(jp_v, jp_hbm.at[wid])
        pltpu.sync_copy(hp_v, hp_hbm.at[wid])

    f = pl.kernel(
        body,
        out_type=(
            jax.ShapeDtypeStruct((NW, B * NBP * NBP), jnp.float32),
            jax.ShapeDtypeStruct((NW, 2 * B * NBP), jnp.float32),
        ),
        mesh=mesh,
        scratch_types=[
            pltpu.VMEM((CHUNK,), jnp.float32),
            pltpu.VMEM((CHUNK,), jnp.float32),
            pltpu.VMEM((4 * B, 128), jnp.float32),
            pltpu.VMEM((B * NBP * NBP,), jnp.float32),
            pltpu.VMEM((2 * B * NBP,), jnp.float32),
        ],
    )
    return f(s, t, stats)


def _finalize_tc(jp, hp, B):
    def body(jp_ref, hp_ref, sd_ref, td_ref, jd_ref):
        jh = jnp.sum(jp_ref[...], axis=0).reshape(B, NBP, NBP)
        hh = jnp.sum(hp_ref[...], axis=0).reshape(2 * B, NBP)
        sh = hh[0:B]
        th = hh[B:2 * B]
        sd_ref[...] = sh / jnp.maximum(
            jnp.sum(sh, axis=-1, keepdims=True), EPS)
        td_ref[...] = th / jnp.maximum(
            jnp.sum(th, axis=-1, keepdims=True), EPS)
        jtot = jnp.sum(jnp.sum(jh, axis=2), axis=1)[:, None, None]
        jd_ref[...] = jh / jnp.maximum(jtot, EPS)

    return pl.pallas_call(
        body,
        out_shape=(
            jax.ShapeDtypeStruct((B, NBP), jnp.float32),
            jax.ShapeDtypeStruct((B, NBP), jnp.float32),
            jax.ShapeDtypeStruct((B, NBP, NBP), jnp.float32),
        ),
    )(jp, hp)


def kernel(source, target):
    B = source.shape[0]
    s = source.reshape(B, -1).astype(jnp.float32)
    t = target.reshape(B, -1).astype(jnp.float32)
    stats = _minmax_tc(s, t)
    jp, hp = _sc_hist(s, t, stats)
    return _finalize_tc(jp, hp, B)


# speedup vs baseline: 293.6108x; 293.6108x over previous
import math

import jax
import jax.numpy as jnp
from jax import lax
from jax.experimental import pallas as pl
from jax.experimental.pallas import tpu as pltpu
from jax.experimental.pallas import tpu_sc as plsc

NUM_BINS = 64
KERNEL_SIGMA = 1.0
KR = math.ceil(2 * KERNEL_SIGMA)
NBP = NUM_BINS + 2 * KR
EPS = 1e-08

NC = 2
NS = 16
L = 16
NW = NC * NS


def _bsp(d):
    ad = jnp.abs(d)
    ad2 = ad * ad
    ad3 = ad2 * ad
    inner = (3.0 * ad3 - 6.0 * ad2 + 4.0) * (1.0 / 6.0)
    e = 2.0 - ad
    outer = e * e * e * (1.0 / 6.0)
    return jnp.where(ad < 1.0, inner, jnp.where(ad < 2.0, outer, 0.0))


def _minmax_tc(s, t):
    B = s.shape[0]

    def body(s_ref, t_ref, o_ref):
        sv = s_ref[...]
        tv = t_ref[...]
        stats = jnp.concatenate(
            [
                jnp.min(sv, axis=1),
                jnp.max(sv, axis=1),
                jnp.min(tv, axis=1),
                jnp.max(tv, axis=1),
            ]
        )
        o_ref[...] = jnp.broadcast_to(stats[:, None], (4 * B, 128))

    return pl.pallas_call(
        body,
        out_shape=jax.ShapeDtypeStruct((4 * B, 128), jnp.float32),
    )(s, t)


def _sc_hist(s, t, stats, B, N):
    CHUNK = N // NW
    VPB = CHUNK // L

    mesh = plsc.VectorSubcoreMesh(core_axis_name="c", subcore_axis_name="s")

    def body(s_hbm, t_hbm, stats_hbm, jp_hbm, hp_hbm,
             s_v, t_v, stats_v, jp_v, hp_v):
        cid = lax.axis_index("c")
        sid = lax.axis_index("s")
        wid = sid * NC + cid

        pltpu.sync_copy(stats_hbm, stats_v)

        zero = jnp.zeros((L,), jnp.float32)

        def zbody(i, c):
            jp_v[pl.ds(i * L, L)] = zero
            return c

        lax.fori_loop(0, (B * NBP * NBP) // L, zbody, 0)

        def zbody2(i, c):
            hp_v[pl.ds(i * L, L)] = zero
            return c

        lax.fori_loop(0, (2 * B * NBP) // L, zbody2, 0)

        for b in range(B):
            base = b * N + wid * CHUNK
            pltpu.sync_copy(s_hbm.at[pl.ds(base, CHUNK)], s_v)
            pltpu.sync_copy(t_hbm.at[pl.ds(base, CHUNK)], t_v)

            smin = stats_v[pl.ds((0 * B + b) * 128, L)]
            smax = stats_v[pl.ds((1 * B + b) * 128, L)]
            tmin = stats_v[pl.ds((2 * B + b) * 128, L)]
            tmax = stats_v[pl.ds((3 * B + b) * 128, L)]
            s_bw = (smax - smin) / NUM_BINS
            s_pmin = smin - s_bw * KR
            t_bw = (tmax - tmin) / NUM_BINS
            t_pmin = tmin - t_bw * KR

            jrow0 = b * NBP * NBP
            hrow_s = b * 2 * NBP
            hrow_t = b * 2 * NBP + NBP

            def vbody(i, c):
                xs = s_v[pl.ds(i * L, L)]
                xt = t_v[pl.ds(i * L, L)]
                bps = (xs - s_pmin) / s_bw
                bpt = (xt - t_pmin) / t_bw
                bis = jnp.clip(bps.astype(jnp.int32), KR, KR + NUM_BINS - 1)
                bit = jnp.clip(bpt.astype(jnp.int32), KR, KR + NUM_BINS - 1)
                bases = bis - (KR - 1)
                baset = bit - (KR - 1)
                fbs = bases.astype(jnp.float32)
                fbt = baset.astype(jnp.float32)
                ws = [_bsp(bps - (fbs + r)) for r in range(2 * KR)]
                wt = [_bsp(bpt - (fbt + r)) for r in range(2 * KR)]
                for r in range(2 * KR):
                    plsc.addupdate_scatter(hp_v, [bases + (hrow_s + r)], ws[r])
                    plsc.addupdate_scatter(hp_v, [baset + (hrow_t + r)], wt[r])
                rowbase = jrow0 + bases * NBP + baset
                for r in range(2 * KR):
                    rb = rowbase + r * NBP
                    for cc in range(2 * KR):
                        plsc.addupdate_scatter(jp_v, [rb + cc], ws[r] * wt[cc])
                return c

            lax.fori_loop(0, VPB, vbody, 0)

        for b in range(B):
            pltpu.sync_copy(
                jp_v.at[pl.ds(b * NBP * NBP, NBP * NBP)],
                jp_hbm.at[pl.ds((b * NW + wid) * NBP * NBP, NBP * NBP)])
            pltpu.sync_copy(
                hp_v.at[pl.ds(b * 2 * NBP, 2 * NBP)],
                hp_hbm.at[pl.ds((b * NW + wid) * 2 * NBP, 2 * NBP)])

    f = pl.kernel(
        body,
        out_type=(
            jax.ShapeDtypeStruct((B * NW * NBP * NBP,), jnp.float32),
            jax.ShapeDtypeStruct((B * NW * 2 * NBP,), jnp.float32),
        ),
        mesh=mesh,
        compiler_params=pltpu.CompilerParams(needs_layout_passes=False),
        scratch_types=[
            pltpu.VMEM((CHUNK,), jnp.float32),
            pltpu.VMEM((CHUNK,), jnp.float32),
            pltpu.VMEM((4 * B * 128,), jnp.float32),
            pltpu.VMEM((B * NBP * NBP,), jnp.float32),
            pltpu.VMEM((2 * B * NBP,), jnp.float32),
        ],
    )
    return f(s, t, stats)


def _finalize_tc(jp, hp, B):
    def body(jp_ref, hp_ref, sd_ref, td_ref, jd_ref):
        jsum = jnp.sum(jp_ref[0], axis=0, keepdims=True)
        hsum = jnp.sum(hp_ref[0], axis=0, keepdims=True)
        sh = hsum[:, 0:NBP]
        th = hsum[:, NBP:2 * NBP]
        sd_ref[0] = sh / jnp.maximum(
            jnp.sum(sh, axis=-1, keepdims=True), EPS)
        td_ref[0] = th / jnp.maximum(
            jnp.sum(th, axis=-1, keepdims=True), EPS)
        jtot = jnp.sum(jsum, axis=-1, keepdims=True)
        jd_ref[0] = jsum / jnp.maximum(jtot, EPS)

    return pl.pallas_call(
        body,
        grid=(B,),
        in_specs=[
            pl.BlockSpec((1, NW, NBP * NBP), lambda b: (b, 0, 0)),
            pl.BlockSpec((1, NW, 2 * NBP), lambda b: (b, 0, 0)),
        ],
        out_specs=(
            pl.BlockSpec((1, 1, NBP), lambda b: (b, 0, 0)),
            pl.BlockSpec((1, 1, NBP), lambda b: (b, 0, 0)),
            pl.BlockSpec((1, 1, NBP * NBP), lambda b: (b, 0, 0)),
        ),
        out_shape=(
            jax.ShapeDtypeStruct((B, 1, NBP), jnp.float32),
            jax.ShapeDtypeStruct((B, 1, NBP), jnp.float32),
            jax.ShapeDtypeStruct((B, 1, NBP * NBP), jnp.float32),
        ),
    )(jp, hp)


def kernel(source, target):
    B = source.shape[0]
    s = source.reshape(B, -1).astype(jnp.float32)
    t = target.reshape(B, -1).astype(jnp.float32)
    N = s.shape[1]
    stats = _minmax_tc(s, t)
    jpf, hpf = _sc_hist(s.reshape(-1), t.reshape(-1), stats.reshape(-1), B, N)
    jp = jpf.reshape(B, NW, NBP * NBP)
    hp = hpf.reshape(B, NW, 2 * NBP)
    sd, td, jdf = _finalize_tc(jp, hp, B)
    return sd.reshape(B, NBP), td.reshape(B, NBP), jdf.reshape(B, NBP, NBP)


# speedup vs baseline: 397.7857x; 1.3548x over previous
import math

import jax
import jax.numpy as jnp
from jax import lax
from jax.experimental import pallas as pl
from jax.experimental.pallas import tpu as pltpu
from jax.experimental.pallas import tpu_sc as plsc

NUM_BINS = 64
KERNEL_SIGMA = 1.0
KR = math.ceil(2 * KERNEL_SIGMA)
NBP = NUM_BINS + 2 * KR
JSZ = NBP * NBP
JPAD = 4736
EPS = 1e-08

NC = 2
NS = 16
L = 16
NW = NC * NS


def _minmax_tc(s, t, B, N):
    def body(s_ref, t_ref, o_ref):
        for b in range(B):
            sv = s_ref[pl.ds(b * N, N)]
            tv = t_ref[pl.ds(b * N, N)]
            vals = (jnp.min(sv), jnp.max(sv), jnp.min(tv), jnp.max(tv))
            for stat, v in enumerate(vals):
                o_ref[pl.ds((stat * B + b) * 128, 128)] = jnp.full(
                    (128,), v, jnp.float32)

    return pl.pallas_call(
        body,
        out_shape=jax.ShapeDtypeStruct((4 * B * 128,), jnp.float32),
    )(s, t)


def _sc_hist(s, t, stats, B, N):
    CHUNK = N // NW
    VPB = CHUNK // L

    mesh = plsc.VectorSubcoreMesh(core_axis_name="c", subcore_axis_name="s")

    def body(s_hbm, t_hbm, stats_hbm, *rest):
        jp_hbms = rest[:B]
        s_v, t_v, stats_v, jp_v = rest[B:]
        cid = lax.axis_index("c")
        sid = lax.axis_index("s")
        wid = sid * NC + cid

        pltpu.sync_copy(stats_hbm, stats_v)

        zero = jnp.zeros((L,), jnp.float32)

        def zbody(i, c):
            jp_v[pl.ds(i * L, L)] = zero
            return c

        lax.fori_loop(0, (B * JPAD) // L, zbody, 0)

        for b in range(B):
            base = b * N + wid * CHUNK
            pltpu.sync_copy(s_hbm.at[pl.ds(base, CHUNK)], s_v)
            pltpu.sync_copy(t_hbm.at[pl.ds(base, CHUNK)], t_v)

            smin = stats_v[pl.ds((0 * B + b) * 128, L)]
            smax = stats_v[pl.ds((1 * B + b) * 128, L)]
            tmin = stats_v[pl.ds((2 * B + b) * 128, L)]
            tmax = stats_v[pl.ds((3 * B + b) * 128, L)]
            s_bw = (smax - smin) / NUM_BINS
            s_pmin = smin - s_bw * KR
            t_bw = (tmax - tmin) / NUM_BINS
            t_pmin = tmin - t_bw * KR

            jrow0 = b * JPAD

            def vbody(i, c):
                xs = s_v[pl.ds(i * L, L)]
                xt = t_v[pl.ds(i * L, L)]
                bps = (xs - s_pmin) / s_bw
                bpt = (xt - t_pmin) / t_bw
                bis = jnp.clip(bps.astype(jnp.int32), KR, KR + NUM_BINS - 1)
                bit = jnp.clip(bpt.astype(jnp.int32), KR, KR + NUM_BINS - 1)
                us = bps - bis.astype(jnp.float32)
                ut = bpt - bit.astype(jnp.float32)

                def taps(u):
                    v = 1.0 - u
                    u2 = u * u
                    u3 = u2 * u
                    v2 = v * v
                    v3 = v2 * v
                    c23 = 2.0 / 3.0
                    return (v3 * (1.0 / 6.0),
                            0.5 * u3 - u2 + c23,
                            0.5 * v3 - v2 + c23,
                            u3 * (1.0 / 6.0))

                ws = taps(us)
                wt = taps(ut)
                rowbase = jrow0 + (bis - 1) * NBP + (bit - 1)
                for r in range(2 * KR):
                    for cc in range(2 * KR):
                        plsc.addupdate_scatter(
                            jp_v, [rowbase + (r * NBP + cc)], ws[r] * wt[cc])
                return c

            lax.fori_loop(0, VPB, vbody, 0)

        for b in range(B):
            pltpu.sync_copy(jp_v.at[pl.ds(b * JPAD, JPAD)],
                            jp_hbms[b].at[pl.ds(wid * JPAD, JPAD)])

    f = pl.kernel(
        body,
        out_type=tuple(
            jax.ShapeDtypeStruct((NW * JPAD,), jnp.float32) for _ in range(B)
        ),
        mesh=mesh,
        compiler_params=pltpu.CompilerParams(needs_layout_passes=False),
        scratch_types=[
            pltpu.VMEM((CHUNK,), jnp.float32),
            pltpu.VMEM((CHUNK,), jnp.float32),
            pltpu.VMEM((4 * B * 128,), jnp.float32),
            pltpu.VMEM((B * JPAD,), jnp.float32),
        ],
    )
    return f(s, t, stats)


def _finalize_tc(jps):
    B = len(jps)

    def body(*refs):
        jp_refs = refs[:B]
        sd_refs = refs[B:2 * B]
        td_refs = refs[2 * B:3 * B]
        jd_refs = refs[3 * B:4 * B]
        kk = jax.lax.broadcasted_iota(jnp.int32, (JPAD, 128), 0)
        ii = jax.lax.broadcasted_iota(jnp.int32, (JPAD, 128), 1)
        rmat = ((kk // NBP) == ii).astype(jnp.float32)
        cmat = ((kk % NBP) == ii).astype(jnp.float32)
        for b in range(B):
            acc = jp_refs[b][pl.ds(0, JPAD)]
            for w in range(1, NW):
                acc = acc + jp_refs[b][pl.ds(w * JPAD, JPAD)]
            js = acc[None, :]
            sh = jnp.dot(js, rmat, preferred_element_type=jnp.float32)
            th = jnp.dot(js, cmat, preferred_element_type=jnp.float32)
            sd_refs[b][...] = (sh / jnp.maximum(
                jnp.sum(sh, axis=-1, keepdims=True), EPS)).reshape(128)
            td_refs[b][...] = (th / jnp.maximum(
                jnp.sum(th, axis=-1, keepdims=True), EPS)).reshape(128)
            jtot = jnp.sum(js, axis=-1, keepdims=True)
            jd_refs[b][...] = (js / jnp.maximum(jtot, EPS)).reshape(JPAD)

    out = pl.pallas_call(
        body,
        out_shape=(
            tuple(jax.ShapeDtypeStruct((128,), jnp.float32) for _ in range(B))
            + tuple(jax.ShapeDtypeStruct((128,), jnp.float32) for _ in range(B))
            + tuple(jax.ShapeDtypeStruct((JPAD,), jnp.float32) for _ in range(B))
        ),
    )(*jps)
    return out[:B], out[B:2 * B], out[2 * B:3 * B]


def kernel(source, target):
    B = source.shape[0]
    s = source.reshape(-1).astype(jnp.float32)
    t = target.reshape(-1).astype(jnp.float32)
    N = s.shape[0] // B
    stats = _minmax_tc(s, t, B, N)
    jps = _sc_hist(s, t, stats, B, N)
    sds, tds, jds = _finalize_tc(jps)
    sd = jnp.stack(sds)[:, :NBP]
    td = jnp.stack(tds)[:, :NBP]
    jd = jnp.stack(jds)[:, :JSZ].reshape(B, NBP, NBP)
    return sd, td, jd


# speedup vs baseline: 463.6600x; 1.1656x over previous
import math

import jax
import jax.numpy as jnp
from jax import lax
from jax.experimental import pallas as pl
from jax.experimental.pallas import tpu as pltpu
from jax.experimental.pallas import tpu_sc as plsc

NUM_BINS = 64
KERNEL_SIGMA = 1.0
KR = math.ceil(2 * KERNEL_SIGMA)
NBP = NUM_BINS + 2 * KR
JSZ = NBP * NBP
JPAD = 4736
EPS = 1e-08

NC = 2
NS = 16
L = 16
NW = NC * NS


def _minmax_tc(s, t, B, N):
    def body(s_ref, t_ref, o_ref):
        for b in range(B):
            sv = s_ref[pl.ds(b * N, N)]
            tv = t_ref[pl.ds(b * N, N)]
            vals = (jnp.min(sv), jnp.max(sv), jnp.min(tv), jnp.max(tv))
            for stat, v in enumerate(vals):
                o_ref[pl.ds((stat * B + b) * 128, 128)] = jnp.full(
                    (128,), v, jnp.float32)

    return pl.pallas_call(
        body,
        out_shape=jax.ShapeDtypeStruct((4 * B * 128,), jnp.float32),
    )(s, t)


def _sc_hist(s, t, stats, B, N):
    CHUNK = N // NW
    VPB = CHUNK // L

    mesh = plsc.VectorSubcoreMesh(core_axis_name="c", subcore_axis_name="s")

    def body(s_hbm, t_hbm, stats_hbm, *rest):
        jp_hbms = rest[:B]
        s_v, t_v, stats_v, jp_v = rest[B:]
        cid = lax.axis_index("c")
        sid = lax.axis_index("s")
        wid = sid * NC + cid

        pltpu.sync_copy(stats_hbm, stats_v)

        zero = jnp.zeros((L,), jnp.float32)

        def zbody(i, c):
            jp_v[pl.ds(i * L, L)] = zero
            return c

        lax.fori_loop(0, (B * JPAD) // L, zbody, 0)

        for b in range(B):
            base = b * N + wid * CHUNK
            pltpu.sync_copy(s_hbm.at[pl.ds(base, CHUNK)], s_v)
            pltpu.sync_copy(t_hbm.at[pl.ds(base, CHUNK)], t_v)

            smin = stats_v[pl.ds((0 * B + b) * 128, L)]
            smax = stats_v[pl.ds((1 * B + b) * 128, L)]
            tmin = stats_v[pl.ds((2 * B + b) * 128, L)]
            tmax = stats_v[pl.ds((3 * B + b) * 128, L)]
            s_bw = (smax - smin) / NUM_BINS
            s_pmin = smin - s_bw * KR
            s_inv = 1.0 / s_bw
            t_bw = (tmax - tmin) / NUM_BINS
            t_pmin = tmin - t_bw * KR
            t_inv = 1.0 / t_bw

            jrow0 = b * JPAD

            @plsc.parallel_loop(0, VPB, unroll=4)
            def vbody(i):
                xs = s_v[pl.ds(i * L, L)]
                xt = t_v[pl.ds(i * L, L)]
                bps = (xs - s_pmin) * s_inv
                bpt = (xt - t_pmin) * t_inv
                bis = jnp.clip(bps.astype(jnp.int32), KR, KR + NUM_BINS - 1)
                bit = jnp.clip(bpt.astype(jnp.int32), KR, KR + NUM_BINS - 1)
                us = bps - bis.astype(jnp.float32)
                ut = bpt - bit.astype(jnp.float32)

                def taps(u):
                    v = 1.0 - u
                    u2 = u * u
                    u3 = u2 * u
                    v2 = v * v
                    v3 = v2 * v
                    c23 = 2.0 / 3.0
                    return (v3 * (1.0 / 6.0),
                            0.5 * u3 - u2 + c23,
                            0.5 * v3 - v2 + c23,
                            u3 * (1.0 / 6.0))

                ws = taps(us)
                wt = taps(ut)
                rowbase = bis * NBP + bit + (jrow0 - NBP - 1)
                for r in range(2 * KR):
                    for cc in range(2 * KR):
                        plsc.addupdate_scatter(
                            jp_v, [rowbase + (r * NBP + cc)], ws[r] * wt[cc])

        for b in range(B):
            pltpu.sync_copy(jp_v.at[pl.ds(b * JPAD, JPAD)],
                            jp_hbms[b].at[pl.ds(wid * JPAD, JPAD)])

    f = pl.kernel(
        body,
        out_type=tuple(
            jax.ShapeDtypeStruct((NW * JPAD,), jnp.float32) for _ in range(B)
        ),
        mesh=mesh,
        compiler_params=pltpu.CompilerParams(needs_layout_passes=False),
        scratch_types=[
            pltpu.VMEM((CHUNK,), jnp.float32),
            pltpu.VMEM((CHUNK,), jnp.float32),
            pltpu.VMEM((4 * B * 128,), jnp.float32),
            pltpu.VMEM((B * JPAD,), jnp.float32),
        ],
    )
    return f(s, t, stats)


def _finalize_tc(jps):
    B = len(jps)

    def body(*refs):
        jp_refs = refs[:B]
        sd_ref, td_ref, jd_ref = refs[B:]
        kk = jax.lax.broadcasted_iota(jnp.int32, (JPAD, 128), 0)
        ii = jax.lax.broadcasted_iota(jnp.int32, (JPAD, 128), 1)
        rmat = ((kk // NBP) == ii).astype(jnp.float32)
        cmat = ((kk % NBP) == ii).astype(jnp.float32)
        for b in range(B):
            acc = jp_refs[b][pl.ds(0, JPAD)]
            for w in range(1, NW):
                acc = acc + jp_refs[b][pl.ds(w * JPAD, JPAD)]
            js = acc[None, :]
            sh = jnp.dot(js, rmat, preferred_element_type=jnp.float32)
            th = jnp.dot(js, cmat, preferred_element_type=jnp.float32)
            sd_ref[b] = (sh / jnp.maximum(
                jnp.sum(sh, axis=-1, keepdims=True), EPS))[0, :NBP]
            td_ref[b] = (th / jnp.maximum(
                jnp.sum(th, axis=-1, keepdims=True), EPS))[0, :NBP]
            jtot = jnp.sum(js, axis=-1, keepdims=True)
            jd_ref[b] = (js / jnp.maximum(jtot, EPS))[0, :JSZ]

    return pl.pallas_call(
        body,
        out_shape=(
            jax.ShapeDtypeStruct((B, NBP), jnp.float32),
            jax.ShapeDtypeStruct((B, NBP), jnp.float32),
            jax.ShapeDtypeStruct((B, JSZ), jnp.float32),
        ),
    )(*jps)


def kernel(source, target):
    B = source.shape[0]
    s = source.reshape(-1).astype(jnp.float32)
    t = target.reshape(-1).astype(jnp.float32)
    N = s.shape[0] // B
    stats = _minmax_tc(s, t, B, N)
    jps = _sc_hist(s, t, stats, B, N)
    sd, td, jd = _finalize_tc(jps)
    return sd, td, jd.reshape(B, NBP, NBP)


# speedup vs baseline: 508.5068x; 1.0967x over previous
import math

import jax
import jax.numpy as jnp
from jax import lax
from jax.experimental import pallas as pl
from jax.experimental.pallas import tpu as pltpu
from jax.experimental.pallas import tpu_sc as plsc

NUM_BINS = 64
KERNEL_SIGMA = 1.0
KR = math.ceil(2 * KERNEL_SIGMA)
NBP = NUM_BINS + 2 * KR
JSZ = NBP * NBP
JPAD = 4736
EPS = 1e-08

NC = 2
NS = 16
L = 16
NW = NC * NS


def _minmax_tc(s, t, B, N):
    G = 4 * B
    CPB = G // B
    CH = N // CPB

    def body(s_ref, t_ref, o_ref):
        g = pl.program_id(0)
        b = g // CPB
        first = (g % CPB) == 0
        sv = s_ref[...]
        tv = t_ref[...]
        vals = (jnp.min(sv), jnp.max(sv), jnp.min(tv), jnp.max(tv))
        for stat, v in enumerate(vals):
            idx = pl.ds(stat * B * 128 + b * 128, 128)
            vvec = jnp.full((128,), v, jnp.float32)
            comb = jnp.minimum if stat % 2 == 0 else jnp.maximum

            @pl.when(first)
            def _(idx=idx, vvec=vvec):
                o_ref[idx] = vvec

            @pl.when(jnp.logical_not(first))
            def _(idx=idx, vvec=vvec, comb=comb):
                o_ref[idx] = comb(o_ref[idx], vvec)

    return pl.pallas_call(
        body,
        grid=(G,),
        in_specs=[
            pl.BlockSpec((CH,), lambda g: (g,)),
            pl.BlockSpec((CH,), lambda g: (g,)),
        ],
        out_specs=pl.BlockSpec((4 * B * 128,), lambda g: (0,)),
        out_shape=jax.ShapeDtypeStruct((4 * B * 128,), jnp.float32),
    )(s, t)


def _sc_hist(s, t, stats, B, N):
    CHUNK = N // NW
    VPB = CHUNK // L

    mesh = plsc.VectorSubcoreMesh(core_axis_name="c", subcore_axis_name="s")

    def body(s_hbm, t_hbm, stats_hbm, *rest):
        jp_hbms = rest[:B]
        s_v, t_v, stats_v, jp_v = rest[B:]
        cid = lax.axis_index("c")
        sid = lax.axis_index("s")
        wid = sid * NC + cid

        pltpu.sync_copy(stats_hbm, stats_v)

        zero = jnp.zeros((L,), jnp.float32)

        def zbody(i, c):
            jp_v[pl.ds(i * L, L)] = zero
            return c

        lax.fori_loop(0, (B * JPAD) // L, zbody, 0)

        for b in range(B):
            base = b * N + wid * CHUNK
            pltpu.sync_copy(s_hbm.at[pl.ds(base, CHUNK)], s_v)
            pltpu.sync_copy(t_hbm.at[pl.ds(base, CHUNK)], t_v)

            smin = stats_v[pl.ds((0 * B + b) * 128, L)]
            smax = stats_v[pl.ds((1 * B + b) * 128, L)]
            tmin = stats_v[pl.ds((2 * B + b) * 128, L)]
            tmax = stats_v[pl.ds((3 * B + b) * 128, L)]
            s_bw = (smax - smin) / NUM_BINS
            s_pmin = smin - s_bw * KR
            s_inv = 1.0 / s_bw
            t_bw = (tmax - tmin) / NUM_BINS
            t_pmin = tmin - t_bw * KR
            t_inv = 1.0 / t_bw

            jrow0 = b * JPAD

            @plsc.parallel_loop(0, VPB, unroll=8)
            def vbody(i):
                xs = s_v[pl.ds(i * L, L)]
                xt = t_v[pl.ds(i * L, L)]
                bps = (xs - s_pmin) * s_inv
                bpt = (xt - t_pmin) * t_inv
                bis = jnp.clip(bps.astype(jnp.int32), KR, KR + NUM_BINS - 1)
                bit = jnp.clip(bpt.astype(jnp.int32), KR, KR + NUM_BINS - 1)
                us = bps - bis.astype(jnp.float32)
                ut = bpt - bit.astype(jnp.float32)

                def taps(u):
                    v = 1.0 - u
                    u2 = u * u
                    u3 = u2 * u
                    v2 = v * v
                    v3 = v2 * v
                    c23 = 2.0 / 3.0
                    return (v3 * (1.0 / 6.0),
                            0.5 * u3 - u2 + c23,
                            0.5 * v3 - v2 + c23,
                            u3 * (1.0 / 6.0))

                ws = taps(us)
                wt = taps(ut)
                rowbase = bis * NBP + bit + (jrow0 - NBP - 1)
                for r in range(2 * KR):
                    for cc in range(2 * KR):
                        plsc.addupdate_scatter(
                            jp_v, [rowbase + (r * NBP + cc)], ws[r] * wt[cc])

        for b in range(B):
            pltpu.sync_copy(jp_v.at[pl.ds(b * JPAD, JPAD)],
                            jp_hbms[b].at[pl.ds(wid * JPAD, JPAD)])

    f = pl.kernel(
        body,
        out_type=tuple(
            jax.ShapeDtypeStruct((NW * JPAD,), jnp.float32) for _ in range(B)
        ),
        mesh=mesh,
        compiler_params=pltpu.CompilerParams(needs_layout_passes=False),
        scratch_types=[
            pltpu.VMEM((CHUNK,), jnp.float32),
            pltpu.VMEM((CHUNK,), jnp.float32),
            pltpu.VMEM((4 * B * 128,), jnp.float32),
            pltpu.VMEM((B * JPAD,), jnp.float32),
        ],
    )
    return f(s, t, stats)


def _finalize_tc(jps):
    B = len(jps)

    def body(*refs):
        jp_refs = refs[:B]
        sd_ref, td_ref, jd_ref = refs[B:]
        kk = jax.lax.broadcasted_iota(jnp.int32, (JPAD, 128), 0)
        ii = jax.lax.broadcasted_iota(jnp.int32, (JPAD, 128), 1)
        rmat = ((kk // NBP) == ii).astype(jnp.float32)
        cmat = ((kk % NBP) == ii).astype(jnp.float32)
        for b in range(B):
            acc = jp_refs[b][pl.ds(0, JPAD)]
            for w in range(1, NW):
                acc = acc + jp_refs[b][pl.ds(w * JPAD, JPAD)]
            js = acc[None, :]
            sh = jnp.dot(js, rmat, preferred_element_type=jnp.float32)
            th = jnp.dot(js, cmat, preferred_element_type=jnp.float32)
            sd_ref[b] = (sh / jnp.maximum(
                jnp.sum(sh, axis=-1, keepdims=True), EPS))[0, :NBP]
            td_ref[b] = (th / jnp.maximum(
                jnp.sum(th, axis=-1, keepdims=True), EPS))[0, :NBP]
            jtot = jnp.sum(js, axis=-1, keepdims=True)
            jd_ref[b] = (js / jnp.maximum(jtot, EPS))[0, :JSZ]

    return pl.pallas_call(
        body,
        out_shape=(
            jax.ShapeDtypeStruct((B, NBP), jnp.float32),
            jax.ShapeDtypeStruct((B, NBP), jnp.float32),
            jax.ShapeDtypeStruct((B, JSZ), jnp.float32),
        ),
    )(*jps)


def kernel(source, target):
    B = source.shape[0]
    perm = (0, 1, 2, 4, 3)
    s = source.transpose(perm).reshape(-1).astype(jnp.float32)
    t = target.transpose(perm).reshape(-1).astype(jnp.float32)
    N = s.shape[0] // B
    stats = _minmax_tc(s, t, B, N)
    jps = _sc_hist(s, t, stats, B, N)
    sd, td, jd = _finalize_tc(jps)
    return sd, td, jd.reshape(B, NBP, NBP)


# speedup vs baseline: 593.1921x; 1.1665x over previous
import math

import jax
import jax.numpy as jnp
from jax import lax
from jax.experimental import pallas as pl
from jax.experimental.pallas import tpu as pltpu
from jax.experimental.pallas import tpu_sc as plsc

NUM_BINS = 64
KERNEL_SIGMA = 1.0
KR = math.ceil(2 * KERNEL_SIGMA)
NBP = NUM_BINS + 2 * KR
JSZ = NBP * NBP
JPAD = 4736
EPS = 1e-08

NC = 2
NS = 16
L = 16

INF = float("inf")


def _sc_hist(s, t, B, N):
    assert B == NC
    CHUNK = N // NS
    VPB = CHUNK // L

    mesh = plsc.VectorSubcoreMesh(core_axis_name="c", subcore_axis_name="s")

    def body(s_hbm, t_hbm, *rest):
        jp_hbms = rest[:B]
        s_v, t_v, jp_v, pub_v, all_v, shared = rest[B:]
        cid = lax.axis_index("c")
        sid = lax.axis_index("s")

        base = cid * N + sid * CHUNK
        pltpu.sync_copy(s_hbm.at[pl.ds(base, CHUNK)], s_v)
        pltpu.sync_copy(t_hbm.at[pl.ds(base, CHUNK)], t_v)

        def mbody(i, carry):
            smn, smx, tmn, tmx = carry
            xs = s_v[pl.ds(i * L, L)]
            xt = t_v[pl.ds(i * L, L)]
            return (jnp.minimum(smn, xs), jnp.maximum(smx, xs),
                    jnp.minimum(tmn, xt), jnp.maximum(tmx, xt))

        ival = (jnp.full((L,), INF), jnp.full((L,), -INF),
                jnp.full((L,), INF), jnp.full((L,), -INF))
        smn, smx, tmn, tmx = lax.fori_loop(0, VPB, mbody, ival)

        lane = jax.lax.broadcasted_iota(jnp.int32, (L,), 0)
        pub = jnp.where(
            lane == 0, jnp.min(smn),
            jnp.where(lane == 1, -jnp.max(smx),
                      jnp.where(lane == 2, jnp.min(tmn),
                                jnp.where(lane == 3, -jnp.max(tmx), INF))))
        pub_v[...] = pub
        pltpu.sync_copy(pub_v, shared.at[sid])
        plsc.subcore_barrier()
        pltpu.sync_copy(shared, all_v)

        m = all_v[0, pl.ds(0, L)]
        for r in range(1, NS):
            m = jnp.minimum(m, all_v[r, pl.ds(0, L)])
        zvec = jnp.zeros((L,), jnp.float32)
        s_min = zvec + jnp.min(jnp.where(lane == 0, m, INF))
        s_max = zvec - jnp.min(jnp.where(lane == 1, m, INF))
        t_min = zvec + jnp.min(jnp.where(lane == 2, m, INF))
        t_max = zvec - jnp.min(jnp.where(lane == 3, m, INF))

        s_bw = (s_max - s_min) / NUM_BINS
        s_pmin = s_min - s_bw * KR
        s_inv = 1.0 / s_bw
        t_bw = (t_max - t_min) / NUM_BINS
        t_pmin = t_min - t_bw * KR
        t_inv = 1.0 / t_bw

        zero = jnp.zeros((L,), jnp.float32)

        def zbody(i, c):
            jp_v[pl.ds(i * L, L)] = zero
            return c

        lax.fori_loop(0, JPAD // L, zbody, 0)

        @plsc.parallel_loop(0, VPB, unroll=8)
        def vbody(i):
            xs = s_v[pl.ds(i * L, L)]
            xt = t_v[pl.ds(i * L, L)]
            bps = (xs - s_pmin) * s_inv
            bpt = (xt - t_pmin) * t_inv
            bis = jnp.clip(bps.astype(jnp.int32), KR, KR + NUM_BINS - 1)
            bit = jnp.clip(bpt.astype(jnp.int32), KR, KR + NUM_BINS - 1)
            us = bps - bis.astype(jnp.float32)
            ut = bpt - bit.astype(jnp.float32)

            def taps(u):
                v = 1.0 - u
                u2 = u * u
                u3 = u2 * u
                v2 = v * v
                v3 = v2 * v
                c23 = 2.0 / 3.0
                return (v3 * (1.0 / 6.0),
                        0.5 * u3 - u2 + c23,
                        0.5 * v3 - v2 + c23,
                        u3 * (1.0 / 6.0))

            ws = taps(us)
            wt = taps(ut)
            rowbase = bis * NBP + bit - (NBP + 1)
            for r in range(2 * KR):
                for cc in range(2 * KR):
                    plsc.addupdate_scatter(
                        jp_v, [rowbase + (r * NBP + cc)], ws[r] * wt[cc])

        for b in range(B):
            @pl.when(cid == b)
            def _(b=b):
                pltpu.sync_copy(jp_v,
                                jp_hbms[b].at[pl.ds(sid * JPAD, JPAD)])

    f = pl.kernel(
        body,
        out_type=tuple(
            jax.ShapeDtypeStruct((NS * JPAD,), jnp.float32) for _ in range(B)
        ),
        mesh=mesh,
        compiler_params=pltpu.CompilerParams(needs_layout_passes=False),
        scratch_types=[
            pltpu.VMEM((CHUNK,), jnp.float32),
            pltpu.VMEM((CHUNK,), jnp.float32),
            pltpu.VMEM((JPAD,), jnp.float32),
            pltpu.VMEM((L,), jnp.float32),
            pltpu.VMEM((NS, L), jnp.float32),
            pltpu.VMEM_SHARED((NS, L), jnp.float32),
        ],
    )
    return f(s, t)


def _finalize_tc(jps):
    B = len(jps)

    def body(*refs):
        jp_refs = refs[:B]
        sd_ref, td_ref, jd_ref = refs[B:]
        kk = jax.lax.broadcasted_iota(jnp.int32, (JPAD, 128), 0)
        ii = jax.lax.broadcasted_iota(jnp.int32, (JPAD, 128), 1)
        rmat = ((kk // NBP) == ii).astype(jnp.float32)
        cmat = ((kk % NBP) == ii).astype(jnp.float32)
        for b in range(B):
            acc = jp_refs[b][pl.ds(0, JPAD)]
            for w in range(1, NS):
                acc = acc + jp_refs[b][pl.ds(w * JPAD, JPAD)]
            js = acc[None, :]
            sh = jnp.dot(js, rmat, preferred_element_type=jnp.float32)
            th = jnp.dot(js, cmat, preferred_element_type=jnp.float32)
            sd_ref[b] = (sh / jnp.maximum(
                jnp.sum(sh, axis=-1, keepdims=True), EPS))[0, :NBP]
            td_ref[b] = (th / jnp.maximum(
                jnp.sum(th, axis=-1, keepdims=True), EPS))[0, :NBP]
            jtot = jnp.sum(js, axis=-1, keepdims=True)
            jd_ref[b] = (js / jnp.maximum(jtot, EPS))[0, :JSZ]

    return pl.pallas_call(
        body,
        out_shape=(
            jax.ShapeDtypeStruct((B, NBP), jnp.float32),
            jax.ShapeDtypeStruct((B, NBP), jnp.float32),
            jax.ShapeDtypeStruct((B, JSZ), jnp.float32),
        ),
    )(*jps)


def kernel(source, target):
    B = source.shape[0]
    perm = (0, 1, 2, 4, 3)
    s = source.transpose(perm).reshape(-1).astype(jnp.float32)
    t = target.transpose(perm).reshape(-1).astype(jnp.float32)
    N = s.shape[0] // B
    jps = _sc_hist(s, t, B, N)
    sd, td, jd = _finalize_tc(jps)
    return sd, td, jd.reshape(B, NBP, NBP)


# speedup vs baseline: 595.2670x; 1.0035x over previous
import math

import jax
import jax.numpy as jnp
from jax import lax
from jax.experimental import pallas as pl
from jax.experimental.pallas import tpu as pltpu
from jax.experimental.pallas import tpu_sc as plsc

NUM_BINS = 64
KERNEL_SIGMA = 1.0
KR = math.ceil(2 * KERNEL_SIGMA)
NBP = NUM_BINS + 2 * KR
JSZ = NBP * NBP
JPAD = 4736
EPS = 1e-08

NC = 2
NS = 16
L = 16

INF = float("inf")


def _sc_hist(s, t, B, N):
    assert B == NC
    CHUNK = N // NS
    VPB = CHUNK // L

    mesh = plsc.VectorSubcoreMesh(core_axis_name="c", subcore_axis_name="s")

    def body(s_hbm, t_hbm, *rest):
        jp_hbms = rest[:B]
        s_v, t_v, jp_v, pub_v, all_v, shared = rest[B:]
        cid = lax.axis_index("c")
        sid = lax.axis_index("s")

        base = cid * N + sid * CHUNK
        pltpu.sync_copy(s_hbm.at[pl.ds(base, CHUNK)], s_v)
        pltpu.sync_copy(t_hbm.at[pl.ds(base, CHUNK)], t_v)

        def mbody(i, carry):
            smn, smx, tmn, tmx = carry
            xs = s_v[pl.ds(i * L, L)]
            xt = t_v[pl.ds(i * L, L)]
            return (jnp.minimum(smn, xs), jnp.maximum(smx, xs),
                    jnp.minimum(tmn, xt), jnp.maximum(tmx, xt))

        ival = (jnp.full((L,), INF), jnp.full((L,), -INF),
                jnp.full((L,), INF), jnp.full((L,), -INF))
        smn, smx, tmn, tmx = lax.fori_loop(0, VPB, mbody, ival)

        lane = jax.lax.broadcasted_iota(jnp.int32, (L,), 0)
        pub = jnp.where(
            lane == 0, jnp.min(smn),
            jnp.where(lane == 1, -jnp.max(smx),
                      jnp.where(lane == 2, jnp.min(tmn),
                                jnp.where(lane == 3, -jnp.max(tmx), INF))))
        pub_v[...] = pub
        pltpu.sync_copy(pub_v, shared.at[cid, sid])
        plsc.subcore_barrier()
        pltpu.sync_copy(shared.at[cid], all_v)

        m = all_v[0, pl.ds(0, L)]
        for r in range(1, NS):
            m = jnp.minimum(m, all_v[r, pl.ds(0, L)])
        zvec = jnp.zeros((L,), jnp.float32)
        s_min = zvec + jnp.min(jnp.where(lane == 0, m, INF))
        s_max = zvec - jnp.min(jnp.where(lane == 1, m, INF))
        t_min = zvec + jnp.min(jnp.where(lane == 2, m, INF))
        t_max = zvec - jnp.min(jnp.where(lane == 3, m, INF))

        s_bw = (s_max - s_min) / NUM_BINS
        s_pmin = s_min - s_bw * KR
        s_inv = 1.0 / s_bw
        t_bw = (t_max - t_min) / NUM_BINS
        t_pmin = t_min - t_bw * KR
        t_inv = 1.0 / t_bw

        zero = jnp.zeros((L,), jnp.float32)

        def zbody(i, c):
            jp_v[pl.ds(i * L, L)] = zero
            return c

        lax.fori_loop(0, JPAD // L, zbody, 0)

        @plsc.parallel_loop(0, VPB, unroll=8)
        def vbody(i):
            xs = s_v[pl.ds(i * L, L)]
            xt = t_v[pl.ds(i * L, L)]
            bps = (xs - s_pmin) * s_inv
            bpt = (xt - t_pmin) * t_inv
            bis = jnp.clip(bps.astype(jnp.int32), KR, KR + NUM_BINS - 1)
            bit = jnp.clip(bpt.astype(jnp.int32), KR, KR + NUM_BINS - 1)
            us = bps - bis.astype(jnp.float32)
            ut = bpt - bit.astype(jnp.float32)

            def taps(u):
                v = 1.0 - u
                u2 = u * u
                u3 = u2 * u
                v2 = v * v
                v3 = v2 * v
                c23 = 2.0 / 3.0
                return (v3 * (1.0 / 6.0),
                        0.5 * u3 - u2 + c23,
                        0.5 * v3 - v2 + c23,
                        u3 * (1.0 / 6.0))

            ws = taps(us)
            wt = taps(ut)
            rowbase = bis * NBP + bit - (NBP + 1)
            for r in range(2 * KR):
                for cc in range(2 * KR):
                    plsc.addupdate_scatter(
                        jp_v, [rowbase + (r * NBP + cc)], ws[r] * wt[cc])

        for b in range(B):
            @pl.when(cid == b)
            def _(b=b):
                pltpu.sync_copy(jp_v,
                                jp_hbms[b].at[pl.ds(sid * JPAD, JPAD)])

    f = pl.kernel(
        body,
        out_type=tuple(
            jax.ShapeDtypeStruct((NS * JPAD,), jnp.float32) for _ in range(B)
        ),
        mesh=mesh,
        compiler_params=pltpu.CompilerParams(needs_layout_passes=False),
        scratch_types=[
            pltpu.VMEM((CHUNK,), jnp.float32),
            pltpu.VMEM((CHUNK,), jnp.float32),
            pltpu.VMEM((JPAD,), jnp.float32),
            pltpu.VMEM((L,), jnp.float32),
            pltpu.VMEM((NS, L), jnp.float32),
            pltpu.VMEM_SHARED((NC, NS, L), jnp.float32),
        ],
    )
    return f(s, t)


def _finalize_tc(jps):
    B = len(jps)

    def body(*refs):
        jp_refs = refs[:B]
        sd_ref, td_ref, jd_ref = refs[B:]
        kk = jax.lax.broadcasted_iota(jnp.int32, (JPAD, 128), 0)
        ii = jax.lax.broadcasted_iota(jnp.int32, (JPAD, 128), 1)
        rmat = ((kk // NBP) == ii).astype(jnp.float32)
        cmat = ((kk % NBP) == ii).astype(jnp.float32)
        for b in range(B):
            acc = jp_refs[b][pl.ds(0, JPAD)]
            for w in range(1, NS):
                acc = acc + jp_refs[b][pl.ds(w * JPAD, JPAD)]
            js = acc[None, :]
            sh = jnp.dot(js, rmat, preferred_element_type=jnp.float32)
            th = jnp.dot(js, cmat, preferred_element_type=jnp.float32)
            sd_ref[b] = (sh / jnp.maximum(
                jnp.sum(sh, axis=-1, keepdims=True), EPS))[0, :NBP]
            td_ref[b] = (th / jnp.maximum(
                jnp.sum(th, axis=-1, keepdims=True), EPS))[0, :NBP]
            jtot = jnp.sum(js, axis=-1, keepdims=True)
            jd_ref[b] = (js / jnp.maximum(jtot, EPS))[0, :JSZ]

    return pl.pallas_call(
        body,
        out_shape=(
            jax.ShapeDtypeStruct((B, NBP), jnp.float32),
            jax.ShapeDtypeStruct((B, NBP), jnp.float32),
            jax.ShapeDtypeStruct((B, JSZ), jnp.float32),
        ),
    )(*jps)


def kernel(source, target):
    B = source.shape[0]
    perm = (0, 1, 2, 4, 3)
    s = source.transpose(perm).reshape(-1).astype(jnp.float32)
    t = target.transpose(perm).reshape(-1).astype(jnp.float32)
    N = s.shape[0] // B
    jps = _sc_hist(s, t, B, N)
    sd, td, jd = _finalize_tc(jps)
    return sd, td, jd.reshape(B, NBP, NBP)
